# baseline (speedup 1.0000x reference)
import functools

import jax
import jax.numpy as jnp
from jax import lax
from jax.experimental import pallas as pl
from jax.experimental.pallas import tpu as pltpu
from jax.experimental.pallas import tpu_sc as plsc

BATCH = 16384
EDIM = 64
ANUM = 32
NUM_CORES = 2
NUM_SUBCORES = 16
NW = NUM_CORES * NUM_SUBCORES
BPW = BATCH // NW


def _sc_gather_body(uid_hbm, iid_hbm, eu1_hbm, eu2_hbm,
                    u1_out, u2_out, i1_out, i2_out,
                    uidx_v, iidx_v, buf0, buf1, sem0, sem1):
    c = lax.axis_index("c")
    s = lax.axis_index("s")
    wid = s * NUM_CORES + c
    base = wid * BPW
    pltpu.sync_copy(uid_hbm.at[pl.ds(base, BPW)], uidx_v)
    pltpu.sync_copy(iid_hbm.at[pl.ds(base, BPW)], iidx_v)
    jobs = (
        (eu1_hbm, uidx_v, u1_out),
        (eu2_hbm, uidx_v, u2_out),
        (eu1_hbm, iidx_v, i1_out),
        (eu2_hbm, iidx_v, i2_out),
    )
    bufs = (buf0, buf1)
    sems = (sem0, sem1)
    copies = [None, None]
    copies[0] = pltpu.async_copy(jobs[0][0].at[jobs[0][1]], bufs[0], sems[0])
    for k in range(4):
        if k + 1 < 4:
            nb = (k + 1) % 2
            copies[nb] = pltpu.async_copy(
                jobs[k + 1][0].at[jobs[k + 1][1]], bufs[nb], sems[nb])
        copies[k % 2].wait()
        pltpu.sync_copy(bufs[k % 2], jobs[k][2].at[pl.ds(base, BPW)])


def _sc_gather(user_id, item_id, eu1, eu2):
    mesh = plsc.VectorSubcoreMesh(core_axis_name="c", subcore_axis_name="s")
    row = jax.ShapeDtypeStruct((BATCH, EDIM), jnp.float32)
    fn = pl.kernel(
        _sc_gather_body,
        out_type=(row, row, row, row),
        mesh=mesh,
        scratch_types=(
            pltpu.VMEM((BPW,), jnp.int32),
            pltpu.VMEM((BPW,), jnp.int32),
            pltpu.VMEM((BPW, EDIM), jnp.float32),
            pltpu.VMEM((BPW, EDIM), jnp.float32),
            pltpu.SemaphoreType.DMA,
            pltpu.SemaphoreType.DMA,
        ),
        compiler_params=pltpu.CompilerParams(use_tc_tiling_on_sc=False),
    )
    return fn(user_id, item_id, eu1, eu2)


def _tc_body(u1_ref, u2_ref, i1_ref, i2_ref, w_ref, b_ref,
             prob_ref, pu_ref, pi_ref):
    u1 = u1_ref[...]
    i1 = i1_ref[...]
    w = w_ref[...]
    b = b_ref[...]
    dn = (((1,), (1,)), ((), ()))
    pu_ref[...] = lax.dot_general(u1, w, dn,
                                  preferred_element_type=jnp.float32) + b
    pi_ref[...] = lax.dot_general(i1, w, dn,
                                  preferred_element_type=jnp.float32) + b
    d = jnp.sum(u1 * i1 + u2_ref[...] * i2_ref[...], axis=1, keepdims=True)
    prob_ref[...] = jax.nn.sigmoid(d)


def _tc_compute(u1, u2, i1, i2, w, b):
    blk = 2048
    grid = (BATCH // blk,)
    row_spec = pl.BlockSpec((blk, EDIM), lambda j: (j, 0))
    prob, pu, pi = pl.pallas_call(
        _tc_body,
        grid=grid,
        in_specs=[
            row_spec, row_spec, row_spec, row_spec,
            pl.BlockSpec((ANUM, EDIM), lambda j: (0, 0)),
            pl.BlockSpec((1, ANUM), lambda j: (0, 0)),
        ],
        out_specs=[
            pl.BlockSpec((blk, 1), lambda j: (j, 0)),
            pl.BlockSpec((blk, ANUM), lambda j: (j, 0)),
            pl.BlockSpec((blk, ANUM), lambda j: (j, 0)),
        ],
        out_shape=[
            jax.ShapeDtypeStruct((BATCH, 1), jnp.float32),
            jax.ShapeDtypeStruct((BATCH, ANUM), jnp.float32),
            jax.ShapeDtypeStruct((BATCH, ANUM), jnp.float32),
        ],
    )(u1, u2, i1, i2, w, b.reshape(1, ANUM))
    return prob.reshape(BATCH), pu, pi


def kernel(user_id, item_id, Eu1, Eu2, Ei1, Ei2, W, b):
    u1, u2, i1, i2 = _sc_gather(user_id, item_id, Eu1, Eu2)
    return _tc_compute(u1, u2, i1, i2, W, b)


# speedup vs baseline: 1.1096x; 1.1096x over previous
import jax
import jax.numpy as jnp
from jax import lax
from jax.experimental import pallas as pl
from jax.experimental.pallas import tpu as pltpu
from jax.experimental.pallas import tpu_sc as plsc

BATCH = 16384
EDIM = 64
PDIM = 2 * EDIM
ANUM = 32
ROWS = 100000
NUM_CORES = 2
NUM_SUBCORES = 16
NW = NUM_CORES * NUM_SUBCORES
BPW = BATCH // NW
CHUNK = 256
NCH = BPW // CHUNK


def _repack_body(e1_ref, e2_ref, t_ref):
    t_ref[...] = jnp.concatenate([e1_ref[...], e2_ref[...]], axis=1)


def _repack(eu1, eu2):
    blk = 5000
    return pl.pallas_call(
        _repack_body,
        grid=(ROWS // blk,),
        in_specs=[
            pl.BlockSpec((blk, EDIM), lambda j: (j, 0)),
            pl.BlockSpec((blk, EDIM), lambda j: (j, 0)),
        ],
        out_specs=pl.BlockSpec((blk, PDIM), lambda j: (j, 0)),
        out_shape=jax.ShapeDtypeStruct((ROWS, PDIM), jnp.float32),
    )(eu1, eu2)


def _sc_gather_body(uid_hbm, iid_hbm, t_hbm, u_out, i_out,
                    uidx_v, iidx_v, buf0, buf1, sem0, sem1):
    c = lax.axis_index("c")
    s = lax.axis_index("s")
    wid = s * NUM_CORES + c
    base = wid * BPW
    pltpu.sync_copy(uid_hbm.at[pl.ds(base, BPW)], uidx_v)
    pltpu.sync_copy(iid_hbm.at[pl.ds(base, BPW)], iidx_v)
    jobs = []
    for idx_v, out in ((uidx_v, u_out), (iidx_v, i_out)):
        for ch in range(NCH):
            jobs.append((idx_v, out, ch * CHUNK))
    bufs = (buf0, buf1)
    sems = (sem0, sem1)
    copies = [None, None]
    idx_v, out, off = jobs[0]
    copies[0] = pltpu.async_copy(
        t_hbm.at[idx_v.at[pl.ds(off, CHUNK)]], bufs[0], sems[0])
    for k in range(len(jobs)):
        if k + 1 < len(jobs):
            idx_v, out, off = jobs[k + 1]
            nb = (k + 1) % 2
            copies[nb] = pltpu.async_copy(
                t_hbm.at[idx_v.at[pl.ds(off, CHUNK)]], bufs[nb], sems[nb])
        idx_v, out, off = jobs[k]
        copies[k % 2].wait()
        pltpu.sync_copy(bufs[k % 2], out.at[pl.ds(base + off, CHUNK)])


def _sc_gather(user_id, item_id, packed):
    mesh = plsc.VectorSubcoreMesh(core_axis_name="c", subcore_axis_name="s")
    rows = jax.ShapeDtypeStruct((BATCH, PDIM), jnp.float32)
    fn = pl.kernel(
        _sc_gather_body,
        out_type=(rows, rows),
        mesh=mesh,
        scratch_types=(
            pltpu.VMEM((BPW,), jnp.int32),
            pltpu.VMEM((BPW,), jnp.int32),
            pltpu.VMEM((CHUNK, PDIM), jnp.float32),
            pltpu.VMEM((CHUNK, PDIM), jnp.float32),
            pltpu.SemaphoreType.DMA,
            pltpu.SemaphoreType.DMA,
        ),
        compiler_params=pltpu.CompilerParams(use_tc_tiling_on_sc=False),
    )
    return fn(user_id, item_id, packed)


def _tc_body(u_ref, i_ref, w_ref, b_ref, prob_ref, pu_ref, pi_ref):
    u = u_ref[...]
    i = i_ref[...]
    w = w_ref[...]
    b = b_ref[...]
    dn = (((1,), (1,)), ((), ()))
    pu_ref[...] = lax.dot_general(u[:, :EDIM], w, dn,
                                  preferred_element_type=jnp.float32) + b
    pi_ref[...] = lax.dot_general(i[:, :EDIM], w, dn,
                                  preferred_element_type=jnp.float32) + b
    d = jnp.sum(u * i, axis=1, keepdims=True)
    prob_ref[...] = jax.nn.sigmoid(d)


def _tc_compute(u, i, w, b):
    blk = 2048
    row_spec = pl.BlockSpec((blk, PDIM), lambda j: (j, 0))
    prob, pu, pi = pl.pallas_call(
        _tc_body,
        grid=(BATCH // blk,),
        in_specs=[
            row_spec, row_spec,
            pl.BlockSpec((ANUM, EDIM), lambda j: (0, 0)),
            pl.BlockSpec((1, ANUM), lambda j: (0, 0)),
        ],
        out_specs=[
            pl.BlockSpec((blk, 1), lambda j: (j, 0)),
            pl.BlockSpec((blk, ANUM), lambda j: (j, 0)),
            pl.BlockSpec((blk, ANUM), lambda j: (j, 0)),
        ],
        out_shape=[
            jax.ShapeDtypeStruct((BATCH, 1), jnp.float32),
            jax.ShapeDtypeStruct((BATCH, ANUM), jnp.float32),
            jax.ShapeDtypeStruct((BATCH, ANUM), jnp.float32),
        ],
    )(u, i, w, b.reshape(1, ANUM))
    return prob.reshape(BATCH), pu, pi


def kernel(user_id, item_id, Eu1, Eu2, Ei1, Ei2, W, b):
    packed = _repack(Eu1, Eu2)
    u, i = _sc_gather(user_id, item_id, packed)
    return _tc_compute(u, i, W, b)
